# baseline (speedup 1.0000x reference)
import functools

import jax
import jax.numpy as jnp
from jax import lax
from jax.experimental import pallas as pl
from jax.experimental.pallas import tpu as pltpu
from jax.experimental.pallas import tpu_sc as plsc

_L = 16


def _make_sc_kernel(B, S, D, V):
    NC, NS = 2, 16
    NW = NC * NS
    BC = B // NW
    NB = 4
    LOOKAHEAD = 2
    CH = D // _L
    NG = BC // _L

    mesh = plsc.VectorSubcoreMesh(core_axis_name="c", subcore_axis_name="s")

    scratch = (
        [pltpu.VMEM((BC,), jnp.int32) for _ in range(NB)]
        + [pltpu.VMEM((BC, D), jnp.float32) for _ in range(NB)]
        + [pltpu.VMEM((D, BC), jnp.float32) for _ in range(NB)]
        + [pltpu.VMEM((S, D), jnp.float32),
           pltpu.VMEM((D * _L,), jnp.float32),
           pltpu.VMEM((2, D), jnp.float32)]
        + [pltpu.SemaphoreType.DMA for _ in range(2 * NB)]
    )

    @functools.partial(
        pl.kernel,
        out_type=jax.ShapeDtypeStruct((S, D, B), jnp.float32),
        mesh=mesh,
        scratch_types=scratch,
        compiler_params=pltpu.CompilerParams(use_tc_tiling_on_sc=False,
                                             needs_layout_passes=False),
    )
    def sc_kernel(idx_hbm, tok_hbm, seg_hbm, pos_hbm, out_hbm, *refs):
        idxs = refs[0:NB]
        rows = refs[NB:2 * NB]
        stage = refs[2 * NB:3 * NB]
        posseg = refs[3 * NB]
        dsplat = refs[3 * NB + 1]
        seg_v = refs[3 * NB + 2]
        gsem = refs[3 * NB + 3:3 * NB + 3 + NB]
        osem = refs[3 * NB + 3 + NB:]

        wid = lax.axis_index("s") * NC + lax.axis_index("c")
        b0 = wid * BC
        lane = lax.iota(jnp.int32, _L)

        pltpu.sync_copy(pos_hbm.at[pl.ds(0, S)], posseg)
        pltpu.sync_copy(seg_hbm, seg_v)

        def _posseg_body(j, carry):
            for ci in range(CH):
                sl = pl.ds(ci * _L, _L)
                posseg[j, sl] = posseg[j, sl] + seg_v[1, sl]
            return carry
        lax.fori_loop(0, S, _posseg_body, 0)

        for ci in range(CH):
            sl = pl.ds(ci * _L, _L)
            dch = seg_v[0, sl] - seg_v[1, sl]
            for l in range(_L):
                h = ci * _L + l
                dsplat[pl.ds(h * _L, _L)] = jnp.zeros((_L,), jnp.float32) + dch[l]

        def _issue_gather(c, b):
            pltpu.sync_copy(idx_hbm.at[c, pl.ds(b0, BC)], idxs[b])
            pltpu.async_copy(tok_hbm.at[idxs[b]], rows[b], gsem[b])

        def _wait_gather(b):
            pltpu.make_async_copy(tok_hbm.at[idxs[b]], rows[b], gsem[b]).wait()

        def _out_ref(c):
            return out_hbm.at[c, pl.ds(0, D), pl.ds(b0, BC)]

        def _drain_out(c, b):
            pltpu.make_async_copy(stage[b], _out_ref(c), osem[b]).wait()

        for p in range(LOOKAHEAD):
            _issue_gather(p, p)

        def _chunk_body(it, carry):
            for b in range(NB):
                c = it * NB + b
                _wait_gather(b)

                cvec = [posseg[c, pl.ds(ci * _L, _L)] for ci in range(CH)]

                def _p1(i, c2):
                    for ci in range(CH):
                        plsc.addupdate(rows[b].at[i, pl.ds(ci * _L, _L)],
                                       cvec[ci])
                    return c2
                lax.fori_loop(0, BC, _p1, 0)

                def _fix(g, c2):
                    idx16 = idxs[b][pl.ds(g * _L, _L)]
                    m16 = idx16 == 0
                    cnt = plsc.all_reduce_population_count(m16)[0]

                    @pl.when(cnt > 0)
                    def _slow():
                        g16 = g * _L + lane

                        def _fh(h, c3):
                            col16 = lane * 0 + h
                            plsc.addupdate_scatter(rows[b], [g16, col16],
                                                   dsplat[pl.ds(h * _L, _L)],
                                                   mask=m16)
                            return c3
                        lax.fori_loop(0, D, _fh, 0)
                    return c2
                lax.fori_loop(0, NG, _fix, 0)

                for g in range(NG):
                    g16 = g * _L + lane
                    gsl = pl.ds(g * _L, _L)

                    def _ph(hq, c3, g16=g16, gsl=gsl):
                        for u in range(4):
                            h = hq * 4 + u
                            col16 = lane * 0 + h
                            v = plsc.load_gather(rows[b], [g16, col16])
                            stage[b][h, gsl] = v
                        return c3
                    lax.fori_loop(0, D // 4, _ph, 0)

                pltpu.async_copy(stage[b], _out_ref(c), osem[b])

                nc = c + LOOKAHEAD
                nb2 = (b + LOOKAHEAD) % NB

                @pl.when(nc < S)
                def _ahead():
                    @pl.when(c >= LOOKAHEAD)
                    def _drain():
                        _drain_out(nc - NB, nb2)
                    _issue_gather(nc, nb2)
            return carry
        lax.fori_loop(0, S // NB, _chunk_body, 0)

        for b in range(NB):
            _drain_out(S - NB + b, b)

    return sc_kernel


def kernel(inputs, token_table, segment_table, position_table):
    B, S = inputs.shape
    V, D = token_table.shape
    idxT = inputs.astype(jnp.int32).T
    k = _make_sc_kernel(B, S, D, V)
    out = k(idxT, token_table, segment_table, position_table)
    return out.transpose(2, 0, 1)


# speedup vs baseline: 1.0949x; 1.0949x over previous
import functools

import jax
import jax.numpy as jnp
from jax import lax
from jax.experimental import pallas as pl
from jax.experimental.pallas import tpu as pltpu
from jax.experimental.pallas import tpu_sc as plsc

_L = 16


def _make_sc_kernel(B, S, D, V):
    NC, NS = 2, 16
    NW = NC * NS
    BC = B // NW
    NB = 4
    LOOKAHEAD = 2
    CH = D // _L
    NG = BC // _L
    TH = D // 8
    TB = B // BC
    ST = S // 8

    mesh = plsc.VectorSubcoreMesh(core_axis_name="c", subcore_axis_name="s")

    scratch = (
        [pltpu.VMEM((BC,), jnp.int32) for _ in range(NB)]
        + [pltpu.VMEM((BC, D), jnp.float32) for _ in range(NB)]
        + [pltpu.VMEM((TH, 8, BC), jnp.float32) for _ in range(NB)]
        + [pltpu.VMEM((S, D), jnp.float32),
           pltpu.VMEM((D * _L,), jnp.float32),
           pltpu.VMEM((2, D), jnp.float32)]
        + [pltpu.SemaphoreType.DMA for _ in range(2 * NB)]
    )

    @functools.partial(
        pl.kernel,
        out_type=jax.ShapeDtypeStruct((S, TH, TB, 8, BC), jnp.float32),
        mesh=mesh,
        scratch_types=scratch,
        compiler_params=pltpu.CompilerParams(use_tc_tiling_on_sc=False,
                                             needs_layout_passes=False),
    )
    def sc_kernel(idx_hbm, tok_hbm, seg_hbm, pos_hbm, out_hbm, *refs):
        idxs = refs[0:NB]
        rows = refs[NB:2 * NB]
        stage = refs[2 * NB:3 * NB]
        posseg = refs[3 * NB]
        dsplat = refs[3 * NB + 1]
        seg_v = refs[3 * NB + 2]
        gsem = refs[3 * NB + 3:3 * NB + 3 + NB]
        osem = refs[3 * NB + 3 + NB:]

        wid = lax.axis_index("s") * NC + lax.axis_index("c")
        lane = lax.iota(jnp.int32, _L)

        pltpu.sync_copy(pos_hbm.at[pl.ds(0, S)], posseg)
        pltpu.sync_copy(seg_hbm, seg_v)

        def _posseg_body(j, carry):
            for ci in range(CH):
                sl = pl.ds(ci * _L, _L)
                posseg[j, sl] = posseg[j, sl] + seg_v[1, sl]
            return carry
        lax.fori_loop(0, S, _posseg_body, 0)

        for ci in range(CH):
            sl = pl.ds(ci * _L, _L)
            dch = seg_v[0, sl] - seg_v[1, sl]
            for l in range(_L):
                h = ci * _L + l
                dsplat[pl.ds(h * _L, _L)] = jnp.zeros((_L,), jnp.float32) + dch[l]

        def _issue_gather(c, b):
            pltpu.sync_copy(idx_hbm.at[c // 8, wid, c % 8], idxs[b])
            pltpu.async_copy(tok_hbm.at[idxs[b]], rows[b], gsem[b])

        def _wait_gather(b):
            pltpu.make_async_copy(tok_hbm.at[idxs[b]], rows[b], gsem[b]).wait()

        def _issue_out(c, b):
            for th in range(TH):
                pltpu.async_copy(stage[b].at[th], out_hbm.at[c, th, wid],
                                 osem[b])

        def _drain_out(c, b):
            for th in range(TH):
                pltpu.make_async_copy(stage[b].at[th],
                                      out_hbm.at[c, th, wid], osem[b]).wait()

        for p in range(LOOKAHEAD):
            _issue_gather(p, p)

        def _chunk_body(it, carry):
            for b in range(NB):
                c = it * NB + b
                _wait_gather(b)

                cvec = [posseg[c, pl.ds(ci * _L, _L)] for ci in range(CH)]

                def _p1(i, c2):
                    for ci in range(CH):
                        plsc.addupdate(rows[b].at[i, pl.ds(ci * _L, _L)],
                                       cvec[ci])
                    return c2
                lax.fori_loop(0, BC, _p1, 0)

                def _fix(g, c2):
                    idx16 = idxs[b][pl.ds(g * _L, _L)]
                    m16 = idx16 == 0
                    cnt = plsc.all_reduce_population_count(m16)[0]

                    @pl.when(cnt > 0)
                    def _slow():
                        g16 = g * _L + lane

                        def _fh(h, c3):
                            col16 = lane * 0 + h
                            plsc.addupdate_scatter(rows[b], [g16, col16],
                                                   dsplat[pl.ds(h * _L, _L)],
                                                   mask=m16)
                            return c3
                        lax.fori_loop(0, D, _fh, 0)
                    return c2
                lax.fori_loop(0, NG, _fix, 0)

                for g in range(NG):
                    g16 = g * _L + lane

                    def _pth(th, c3, g16=g16, g=g):
                        for hh in range(8):
                            col16 = lane * 0 + (th * 8 + hh)
                            v = plsc.load_gather(rows[b], [g16, col16])
                            stage[b][th, hh, pl.ds(g * _L, _L)] = v
                        return c3
                    lax.fori_loop(0, TH, _pth, 0)

                _issue_out(c, b)

                nc = c + LOOKAHEAD
                nb2 = (b + LOOKAHEAD) % NB

                @pl.when(nc < S)
                def _ahead():
                    @pl.when(c >= LOOKAHEAD)
                    def _drain():
                        _drain_out(nc - NB, nb2)
                    _issue_gather(nc, nb2)
            return carry
        lax.fori_loop(0, S // NB, _chunk_body, 0)

        for b in range(NB):
            _drain_out(S - NB + b, b)

    return sc_kernel


def kernel(inputs, token_table, segment_table, position_table):
    B, S = inputs.shape
    V, D = token_table.shape
    idx4 = (inputs.astype(jnp.int32).T
            .reshape(S // 8, 8, B // 128, 128).transpose(0, 2, 1, 3))
    k = _make_sc_kernel(B, S, D, V)
    out = k(idx4, token_table, segment_table, position_table)
    return lax.reshape(out, (B, S, D), dimensions=(2, 4, 0, 1, 3))


# speedup vs baseline: 1.7695x; 1.6162x over previous
import functools

import jax
import jax.numpy as jnp
from jax import lax
from jax.experimental import pallas as pl
from jax.experimental.pallas import tpu as pltpu
from jax.experimental.pallas import tpu_sc as plsc

_L = 16


def _make_sc_kernel(B, S, D, V):
    NC, NS = 2, 16
    NW = NC * NS
    BC = B // NW
    NB = 4
    LOOKAHEAD = 2
    CH = D // _L
    NG = BC // _L
    TH = D // 8
    TB = B // BC
    ST = S // 8

    mesh = plsc.VectorSubcoreMesh(core_axis_name="c", subcore_axis_name="s")

    scratch = (
        [pltpu.VMEM((BC,), jnp.int32) for _ in range(NB)]
        + [pltpu.VMEM((BC, D), jnp.float32) for _ in range(NB)]
        + [pltpu.VMEM((TH, 8, BC + 1), jnp.float32) for _ in range(NB)]
        + [pltpu.VMEM((S, D), jnp.float32),
           pltpu.VMEM((D * _L,), jnp.float32),
           pltpu.VMEM((2, D), jnp.float32)]
        + [pltpu.SemaphoreType.DMA for _ in range(2 * NB)]
    )

    @functools.partial(
        pl.kernel,
        out_type=jax.ShapeDtypeStruct((S, TH, TB, 8, BC), jnp.float32),
        mesh=mesh,
        scratch_types=scratch,
        compiler_params=pltpu.CompilerParams(use_tc_tiling_on_sc=False,
                                             needs_layout_passes=False),
    )
    def sc_kernel(idx_hbm, tok_hbm, seg_hbm, pos_hbm, out_hbm, *refs):
        idxs = refs[0:NB]
        rows = refs[NB:2 * NB]
        stage = refs[2 * NB:3 * NB]
        posseg = refs[3 * NB]
        dsplat = refs[3 * NB + 1]
        seg_v = refs[3 * NB + 2]
        gsem = refs[3 * NB + 3:3 * NB + 3 + NB]
        osem = refs[3 * NB + 3 + NB:]

        wid = lax.axis_index("s") * NC + lax.axis_index("c")
        lane = lax.iota(jnp.int32, _L)

        pltpu.sync_copy(pos_hbm.at[pl.ds(0, S)], posseg)
        pltpu.sync_copy(seg_hbm, seg_v)

        def _posseg_body(j, carry):
            for ci in range(CH):
                sl = pl.ds(ci * _L, _L)
                posseg[j, sl] = posseg[j, sl] + seg_v[1, sl]
            return carry
        lax.fori_loop(0, S, _posseg_body, 0)

        for ci in range(CH):
            sl = pl.ds(ci * _L, _L)
            dch = seg_v[0, sl] - seg_v[1, sl]
            for l in range(_L):
                h = ci * _L + l
                dsplat[pl.ds(h * _L, _L)] = jnp.zeros((_L,), jnp.float32) + dch[l]

        def _issue_gather(c, b):
            pltpu.sync_copy(idx_hbm.at[c // 8, wid, c % 8], idxs[b])
            pltpu.async_copy(tok_hbm.at[idxs[b]], rows[b], gsem[b])

        def _wait_gather(b):
            pltpu.make_async_copy(tok_hbm.at[idxs[b]], rows[b], gsem[b]).wait()

        def _issue_out(c, b):
            for th in range(TH):
                pltpu.async_copy(stage[b].at[th, pl.ds(0, 8), pl.ds(0, BC)],
                                 out_hbm.at[c, th, wid], osem[b])

        def _drain_out(c, b):
            for th in range(TH):
                pltpu.make_async_copy(stage[b].at[th, pl.ds(0, 8),
                                                  pl.ds(0, BC)],
                                      out_hbm.at[c, th, wid], osem[b]).wait()

        for p in range(LOOKAHEAD):
            _issue_gather(p, p)

        def _chunk_body(it, carry):
            for b in range(NB):
                c = it * NB + b
                _wait_gather(b)

                cvec = [posseg[c, pl.ds(ci * _L, _L)] for ci in range(CH)]

                def _fix(g, c2):
                    idx16 = idxs[b][pl.ds(g * _L, _L)]
                    m16 = idx16 == 0
                    cnt = plsc.all_reduce_population_count(m16)[0]

                    @pl.when(cnt > 0)
                    def _slow():
                        g16 = g * _L + lane

                        def _fh(h, c3):
                            col16 = lane * 0 + h
                            plsc.addupdate_scatter(rows[b], [g16, col16],
                                                   dsplat[pl.ds(h * _L, _L)],
                                                   mask=m16)
                            return c3
                        lax.fori_loop(0, D, _fh, 0)
                    return c2
                lax.fori_loop(0, NG, _fix, 0)

                th16s = []
                hh16s = []
                for ci in range(CH):
                    h16 = ci * _L + lane
                    th16s.append(h16 >> 3)
                    hh16s.append(h16 & 7)

                def _pm(i, c2):
                    bb16 = lane * 0 + i
                    for ci in range(CH):
                        v = rows[b][i, pl.ds(ci * _L, _L)] + cvec[ci]
                        plsc.store_scatter(stage[b],
                                           [th16s[ci], hh16s[ci], bb16], v)
                    return c2
                lax.fori_loop(0, BC, _pm, 0)

                _issue_out(c, b)

                nc = c + LOOKAHEAD
                nb2 = (b + LOOKAHEAD) % NB

                @pl.when(nc < S)
                def _ahead():
                    @pl.when(c >= LOOKAHEAD)
                    def _drain():
                        _drain_out(nc - NB, nb2)
                    _issue_gather(nc, nb2)
            return carry
        lax.fori_loop(0, S // NB, _chunk_body, 0)

        for b in range(NB):
            _drain_out(S - NB + b, b)

    return sc_kernel


def kernel(inputs, token_table, segment_table, position_table):
    B, S = inputs.shape
    V, D = token_table.shape
    idx4 = (inputs.astype(jnp.int32).T
            .reshape(S // 8, 8, B // 128, 128).transpose(0, 2, 1, 3))
    k = _make_sc_kernel(B, S, D, V)
    out = k(idx4, token_table, segment_table, position_table)
    return lax.reshape(out, (B, S, D), dimensions=(2, 4, 0, 1, 3))


# speedup vs baseline: 1.9571x; 1.1060x over previous
import functools

import jax
import jax.numpy as jnp
from jax import lax
from jax.experimental import pallas as pl
from jax.experimental.pallas import tpu as pltpu
from jax.experimental.pallas import tpu_sc as plsc

_L = 16


def _make_sc_kernel(B, S, D, V):
    NC, NS = 2, 16
    NW = NC * NS
    BC = B // NW
    NB = 4
    LOOKAHEAD = 2
    CH = D // _L
    NG = BC // _L
    TH = D // 8
    TB = B // BC
    ST = S // 8

    mesh = plsc.VectorSubcoreMesh(core_axis_name="c", subcore_axis_name="s")

    scratch = (
        [pltpu.VMEM((BC,), jnp.int32) for _ in range(NB)]
        + [pltpu.VMEM((BC, D), jnp.float32) for _ in range(NB)]
        + [pltpu.VMEM((TH, 8, BC + 1), jnp.float32) for _ in range(NB)]
        + [pltpu.VMEM((S, D), jnp.float32),
           pltpu.VMEM((D * _L,), jnp.float32),
           pltpu.VMEM((2, D), jnp.float32)]
        + [pltpu.SemaphoreType.DMA for _ in range(3 * NB)]
    )

    @functools.partial(
        pl.kernel,
        out_type=jax.ShapeDtypeStruct((S, TH, TB, 8, BC), jnp.float32),
        mesh=mesh,
        scratch_types=scratch,
        compiler_params=pltpu.CompilerParams(use_tc_tiling_on_sc=False,
                                             needs_layout_passes=False),
    )
    def sc_kernel(idx_hbm, tok_hbm, seg_hbm, pos_hbm, out_hbm, *refs):
        idxs = refs[0:NB]
        rows = refs[NB:2 * NB]
        stage = refs[2 * NB:3 * NB]
        posseg = refs[3 * NB]
        dsplat = refs[3 * NB + 1]
        seg_v = refs[3 * NB + 2]
        gsem = refs[3 * NB + 3:3 * NB + 3 + NB]
        osem = refs[3 * NB + 3 + NB:3 * NB + 3 + 2 * NB]
        isem = refs[3 * NB + 3 + 2 * NB:]

        wid = lax.axis_index("s") * NC + lax.axis_index("c")
        lane = lax.iota(jnp.int32, _L)

        pltpu.sync_copy(pos_hbm.at[pl.ds(0, S)], posseg)
        pltpu.sync_copy(seg_hbm, seg_v)

        def _posseg_body(j, carry):
            for ci in range(CH):
                sl = pl.ds(ci * _L, _L)
                posseg[j, sl] = posseg[j, sl] + seg_v[1, sl]
            return carry
        lax.fori_loop(0, S, _posseg_body, 0)

        for ci in range(CH):
            sl = pl.ds(ci * _L, _L)
            dch = seg_v[0, sl] - seg_v[1, sl]
            for l in range(_L):
                h = ci * _L + l
                dsplat[pl.ds(h * _L, _L)] = jnp.zeros((_L,), jnp.float32) + dch[l]

        def _issue_idx(c, b):
            pltpu.async_copy(idx_hbm.at[c // 8, wid, c % 8], idxs[b], isem[b])

        def _wait_idx(c, b):
            pltpu.make_async_copy(idx_hbm.at[c // 8, wid, c % 8], idxs[b],
                                  isem[b]).wait()

        def _issue_gather(c, b):
            _wait_idx(c, b)
            pltpu.async_copy(tok_hbm.at[idxs[b]], rows[b], gsem[b])

        def _wait_gather(b):
            pltpu.make_async_copy(tok_hbm.at[idxs[b]], rows[b], gsem[b]).wait()

        def _issue_out(c, b):
            for th in range(TH):
                pltpu.async_copy(stage[b].at[th, pl.ds(0, 8), pl.ds(0, BC)],
                                 out_hbm.at[c, th, wid], osem[b])

        def _drain_out(c, b):
            for th in range(TH):
                pltpu.make_async_copy(stage[b].at[th, pl.ds(0, 8),
                                                  pl.ds(0, BC)],
                                      out_hbm.at[c, th, wid], osem[b]).wait()

        for p in range(LOOKAHEAD + 1):
            _issue_idx(p, p)
        for p in range(LOOKAHEAD):
            _issue_gather(p, p)

        def _chunk_body(it, carry):
            for b in range(NB):
                c = it * NB + b
                nc3 = c + LOOKAHEAD + 1
                b3 = (b + LOOKAHEAD + 1) % NB

                @pl.when(nc3 < S)
                def _ahead_idx():
                    _issue_idx(nc3, b3)

                _wait_gather(b)

                cvec = [posseg[c, pl.ds(ci * _L, _L)] for ci in range(CH)]

                def _fix(g, c2):
                    idx16 = idxs[b][pl.ds(g * _L, _L)]
                    m16 = idx16 == 0
                    cnt = plsc.all_reduce_population_count(m16)[0]

                    @pl.when(cnt > 0)
                    def _slow():
                        g16 = g * _L + lane

                        def _fh(h, c3):
                            col16 = lane * 0 + h
                            plsc.addupdate_scatter(rows[b], [g16, col16],
                                                   dsplat[pl.ds(h * _L, _L)],
                                                   mask=m16)
                            return c3
                        lax.fori_loop(0, D, _fh, 0)
                    return c2
                lax.fori_loop(0, NG, _fix, 0)

                th16s = []
                hh16s = []
                for ci in range(CH):
                    h16 = ci * _L + lane
                    th16s.append(h16 >> 3)
                    hh16s.append(h16 & 7)

                def _pm(i2, c2):
                    for r in range(2):
                        i = i2 * 2 + r
                        bb16 = lane * 0 + i
                        for ci in range(CH):
                            v = rows[b][i, pl.ds(ci * _L, _L)] + cvec[ci]
                            plsc.store_scatter(stage[b],
                                               [th16s[ci], hh16s[ci], bb16],
                                               v)
                    return c2
                lax.fori_loop(0, BC // 2, _pm, 0)

                _issue_out(c, b)

                nc = c + LOOKAHEAD
                nb2 = (b + LOOKAHEAD) % NB

                @pl.when(nc < S)
                def _ahead():
                    @pl.when(c >= LOOKAHEAD)
                    def _drain():
                        _drain_out(nc - NB, nb2)
                    _issue_gather(nc, nb2)
            return carry
        lax.fori_loop(0, S // NB, _chunk_body, 0)

        for b in range(NB):
            _drain_out(S - NB + b, b)

    return sc_kernel


def kernel(inputs, token_table, segment_table, position_table):
    B, S = inputs.shape
    V, D = token_table.shape
    idx4 = (inputs.astype(jnp.int32).T
            .reshape(S // 8, 8, B // 128, 128).transpose(0, 2, 1, 3))
    k = _make_sc_kernel(B, S, D, V)
    out = k(idx4, token_table, segment_table, position_table)
    return lax.reshape(out, (B, S, D), dimensions=(2, 4, 0, 1, 3))


# speedup vs baseline: 1.9581x; 1.0005x over previous
import functools

import jax
import jax.numpy as jnp
from jax import lax
from jax.experimental import pallas as pl
from jax.experimental.pallas import tpu as pltpu
from jax.experimental.pallas import tpu_sc as plsc

_L = 16


def _make_sc_kernel(B, S, D, V):
    NC, NS = 2, 16
    NW = NC * NS
    BC = B // NW
    NB = 4
    LOOKAHEAD = 2
    CH = D // _L
    NG = BC // _L
    TH = D // 8
    TB = B // BC
    ST = S // 8

    mesh = plsc.VectorSubcoreMesh(core_axis_name="c", subcore_axis_name="s")

    scratch = (
        [pltpu.VMEM((BC,), jnp.int32) for _ in range(NB)]
        + [pltpu.VMEM((BC, D), jnp.float32) for _ in range(NB)]
        + [pltpu.VMEM((TH, 8, BC + 1), jnp.float32) for _ in range(NB)]
        + [pltpu.VMEM((S, D), jnp.float32),
           pltpu.VMEM((D * _L,), jnp.float32),
           pltpu.VMEM((2, D), jnp.float32)]
        + [pltpu.SemaphoreType.DMA for _ in range(3 * NB)]
    )

    @functools.partial(
        pl.kernel,
        out_type=jax.ShapeDtypeStruct((S, TH, TB, 8, BC), jnp.float32),
        mesh=mesh,
        scratch_types=scratch,
        compiler_params=pltpu.CompilerParams(use_tc_tiling_on_sc=False,
                                             needs_layout_passes=False),
    )
    def sc_kernel(idx_hbm, tok_hbm, seg_hbm, pos_hbm, out_hbm, *refs):
        idxs = refs[0:NB]
        rows = refs[NB:2 * NB]
        stage = refs[2 * NB:3 * NB]
        posseg = refs[3 * NB]
        dsplat = refs[3 * NB + 1]
        seg_v = refs[3 * NB + 2]
        gsem = refs[3 * NB + 3:3 * NB + 3 + NB]
        osem = refs[3 * NB + 3 + NB:3 * NB + 3 + 2 * NB]
        isem = refs[3 * NB + 3 + 2 * NB:]

        wid = lax.axis_index("s") * NC + lax.axis_index("c")
        lane = lax.iota(jnp.int32, _L)

        pltpu.sync_copy(pos_hbm.at[pl.ds(0, S)], posseg)
        pltpu.sync_copy(seg_hbm, seg_v)

        def _posseg_body(j, carry):
            for ci in range(CH):
                sl = pl.ds(ci * _L, _L)
                posseg[j, sl] = posseg[j, sl] + seg_v[1, sl]
            return carry
        lax.fori_loop(0, S, _posseg_body, 0)

        for ci in range(CH):
            sl = pl.ds(ci * _L, _L)
            dch = seg_v[0, sl] - seg_v[1, sl]
            for l in range(_L):
                h = ci * _L + l
                dsplat[pl.ds(h * _L, _L)] = jnp.zeros((_L,), jnp.float32) + dch[l]

        def _issue_idx(c, b):
            pltpu.async_copy(idx_hbm.at[c // 8, wid, c % 8], idxs[b], isem[b])

        def _wait_idx(c, b):
            pltpu.make_async_copy(idx_hbm.at[c // 8, wid, c % 8], idxs[b],
                                  isem[b]).wait()

        def _issue_gather(c, b):
            _wait_idx(c, b)
            pltpu.async_copy(tok_hbm.at[idxs[b]], rows[b], gsem[b])

        def _wait_gather(b):
            pltpu.make_async_copy(tok_hbm.at[idxs[b]], rows[b], gsem[b]).wait()

        def _issue_out(c, b):
            for th in range(TH):
                pltpu.async_copy(stage[b].at[th, pl.ds(0, 8), pl.ds(0, BC)],
                                 out_hbm.at[c, th, wid], osem[b])

        def _drain_out(c, b):
            for th in range(TH):
                pltpu.make_async_copy(stage[b].at[th, pl.ds(0, 8),
                                                  pl.ds(0, BC)],
                                      out_hbm.at[c, th, wid], osem[b]).wait()

        for p in range(LOOKAHEAD + 1):
            _issue_idx(p, p)
        for p in range(LOOKAHEAD):
            _issue_gather(p, p)

        def _chunk_body(it, carry):
            for b in range(NB):
                c = it * NB + b
                nc3 = c + LOOKAHEAD + 1
                b3 = (b + LOOKAHEAD + 1) % NB

                @pl.when(nc3 < S)
                def _ahead_idx():
                    _issue_idx(nc3, b3)

                _wait_gather(b)

                cvec = [posseg[c, pl.ds(ci * _L, _L)] for ci in range(CH)]

                def _fix(g, c2):
                    idx16 = idxs[b][pl.ds(g * _L, _L)]
                    m16 = idx16 == 0
                    cnt = plsc.all_reduce_population_count(m16)[0]

                    @pl.when(cnt > 0)
                    def _slow():
                        g16 = g * _L + lane

                        def _fh(h, c3):
                            col16 = lane * 0 + h
                            plsc.addupdate_scatter(rows[b], [g16, col16],
                                                   dsplat[pl.ds(h * _L, _L)],
                                                   mask=m16)
                            return c3
                        lax.fori_loop(0, D, _fh, 0)
                    return c2
                lax.fori_loop(0, NG, _fix, 0)

                th16s = []
                hh16s = []
                for ci in range(CH):
                    h16 = ci * _L + lane
                    th16s.append(h16 >> 3)
                    hh16s.append(h16 & 7)

                def _pm(i2, c2):
                    for r in range(2):
                        i = i2 * 2 + r
                        bb16 = lane * 0 + i
                        for ci in range(CH):
                            v = rows[b][i, pl.ds(ci * _L, _L)] + cvec[ci]
                            plsc.store_scatter(stage[b],
                                               [th16s[ci], hh16s[ci], bb16],
                                               v)
                    return c2
                lax.fori_loop(0, BC // 2, _pm, 0)

                _issue_out(c, b)

                nc = c + LOOKAHEAD
                nb2 = (b + LOOKAHEAD) % NB

                @pl.when(nc < S)
                def _ahead():
                    @pl.when(c >= LOOKAHEAD)
                    def _drain():
                        _drain_out(nc - NB, nb2)
                    _issue_gather(nc, nb2)
            return carry
        lax.fori_loop(0, S // NB, _chunk_body, 0)

        for b in range(NB):
            _drain_out(S - NB + b, b)

    return sc_kernel


def kernel(inputs, token_table, segment_table, position_table):
    B, S = inputs.shape
    V, D = token_table.shape
    idx4 = (inputs.astype(jnp.int32).T
            .reshape(S // 8, 8, B // 128, 128).transpose(0, 2, 1, 3))
    k = _make_sc_kernel(B, S, D, V)
    out = k(idx4, token_table, segment_table, position_table)
    return out.transpose(2, 4, 0, 1, 3).reshape(B, S, D)
